# baseline (speedup 1.0000x reference)
import functools
import numpy as np
import jax
import jax.numpy as jnp
from jax import lax
from jax.experimental import pallas as pl
from jax.experimental.pallas import tpu as pltpu
from jax.experimental.pallas import tpu_sc as plsc

N = 10000
E = 160000
H = 256
NG = 128
NLAYERS = 3
NBLK = 400
NNB = N // NBLK
EBLK = 1000
_DEG_HIST = np.array([0., 1000., 3000., 4000., 1500., 500.])
_b = np.arange(len(_DEG_HIST))
AVG_DEG_LOG = float((np.log(_b + 1.0) * _DEG_HIST).sum() / _DEG_HIST.sum())

NSCW = 32
PBK = 16
BRANGE = N // PBK
ECHUNK = E // NSCW
CAP = 5120
RND = 8
RROWS = 80
DRAIN = 48
NEG_INF = float(np.finfo(np.float32).min)
POS_INF = float(np.finfo(np.float32).max)


def _sc_mesh():
    return plsc.VectorSubcoreMesh(core_axis_name="c", subcore_axis_name="s")


def _bucketize(src, dst):
    @functools.partial(
        pl.kernel, mesh=_sc_mesh(),
        out_type=[jax.ShapeDtypeStruct((PBK, NSCW, CAP), jnp.int32),
                  jax.ShapeDtypeStruct((PBK, NSCW, CAP), jnp.int32),
                  jax.ShapeDtypeStruct((PBK, NSCW, CAP), jnp.int32),
                  jax.ShapeDtypeStruct((NSCW, 16), jnp.int32)],
        scratch_types=[pltpu.VMEM((CAP + 16,), jnp.int32),
                       pltpu.VMEM((CAP + 16,), jnp.int32),
                       pltpu.VMEM((CAP + 16,), jnp.int32),
                       pltpu.VMEM((CAP + 16,), jnp.int32),
                       pltpu.VMEM((CAP + 16,), jnp.int32),
                       pltpu.VMEM((16,), jnp.int32)],
    )
    def kern(src_hbm, dst_hbm, eid_out, src_out, doff_out, cnt_out,
             sbuf, dbuf, oe, os_, od, crow):
        w = lax.axis_index("s") * 2 + lax.axis_index("c")
        base = w * ECHUNK
        lanes = lax.iota(jnp.int32, 16)
        zero = jnp.zeros((16,), jnp.int32)
        pltpu.sync_copy(src_hbm.at[pl.ds(base, ECHUNK)], sbuf.at[pl.ds(0, ECHUNK)])
        pltpu.sync_copy(dst_hbm.at[pl.ds(base, ECHUNK)], dbuf.at[pl.ds(0, ECHUNK)])

        def init(g, _):
            oe[pl.ds(g * 16, 16)] = zero
            os_[pl.ds(g * 16, 16)] = zero
            return 0
        lax.fori_loop(0, (CAP + 16) // 16, init, 0)

        counts = jnp.zeros((16,), jnp.int32)
        for pb in range(PBK):
            lo = pb * BRANGE

            def edge(e, cur):
                dval = dbuf[pl.ds(e, 16)][0]
                sval = sbuf[pl.ds(e, 16)][0]
                ok = (dval >= lo) & (dval < lo + BRANGE)

                @pl.when(ok)
                def _():
                    oe[pl.ds(cur, 16)] = jnp.full((16,), base + e, jnp.int32)
                    os_[pl.ds(cur, 16)] = jnp.full((16,), sval, jnp.int32)
                    od[pl.ds(cur, 16)] = jnp.full((16,), dval - lo, jnp.int32)
                return jnp.where(ok, cur + 1, cur)

            cnt = lax.fori_loop(0, ECHUNK, edge, jnp.int32(0))
            pad = (256 - cnt % 256) % 256

            def padk(k, _):
                od[pl.ds(cnt + k, 16)] = jnp.full((16,), 10000, jnp.int32)
                return 0
            lax.fori_loop(0, pad, padk, 0)
            nch = (cnt + pad) // 256

            def wr(c, _):
                pltpu.sync_copy(oe.at[pl.ds(c * 256, 256)],
                                eid_out.at[pb, w, pl.ds(c * 256, 256)])
                pltpu.sync_copy(os_.at[pl.ds(c * 256, 256)],
                                src_out.at[pb, w, pl.ds(c * 256, 256)])
                pltpu.sync_copy(od.at[pl.ds(c * 256, 256)],
                                doff_out.at[pb, w, pl.ds(c * 256, 256)])
                return 0
            lax.fori_loop(0, nch, wr, 0)
            counts = jnp.where(lanes == pb, cnt, counts)
        crow[...] = counts
        pltpu.sync_copy(crow, cnt_out.at[w])

    return kern(src, dst)


def _seg_stats(table, bidx, bdoff, bcnt, with_deg):
    out_types = [jax.ShapeDtypeStruct((2 * N * H,), jnp.float32)] * 4
    if with_deg:
        out_types = out_types + [jax.ShapeDtypeStruct((2 * N * 16,), jnp.float32)]
    scr = [pltpu.VMEM((NSCW * 16 + 16,), jnp.int32),
           pltpu.VMEM((256 + 16,), jnp.int32),
           pltpu.VMEM((256 + 16,), jnp.int32),
           pltpu.VMEM((DRAIN + 16,), jnp.int32),
           pltpu.VMEM((DRAIN + 16,), jnp.int32),
           pltpu.VMEM((DRAIN, H), jnp.float32),
           pltpu.VMEM((RROWS * H,), jnp.float32),
           pltpu.VMEM((RROWS * H,), jnp.float32),
           pltpu.VMEM((RROWS * H,), jnp.float32),
           pltpu.VMEM((RROWS * H,), jnp.float32),
           pltpu.VMEM((RROWS * 16,), jnp.float32),
           pltpu.SemaphoreType.DMA]

    @functools.partial(pl.kernel, mesh=_sc_mesh(), out_type=out_types,
                       scratch_types=scr)
    def kern(tab_hbm, idx_hbm, doff_hbm, cnt_hbm, *rest):
        if with_deg:
            (sum_o, min_o, max_o, ssq_o, deg_o, cbuf, ibuf, dbuf, ci, cd,
             rows, asum, amin, amax, assq, adeg, sem) = rest
        else:
            deg_o = None
            (sum_o, min_o, max_o, ssq_o, cbuf, ibuf, dbuf, ci, cd,
             rows, asum, amin, amax, assq, adeg, sem) = rest
        w = lax.axis_index("s") * 2 + lax.axis_index("c")
        b = w // 2
        hh = w % 2
        zf = jnp.zeros((16,), jnp.float32)
        onef = jnp.ones((16,), jnp.float32)
        pinf = jnp.full((16,), POS_INF, jnp.float32)
        ninf = jnp.full((16,), NEG_INF, jnp.float32)
        pltpu.sync_copy(cnt_hbm.at[pl.ds(0, NSCW * 16)], cbuf.at[pl.ds(0, NSCW * 16)])
        ci[pl.ds(0, 16)] = jnp.zeros((16,), jnp.int32)
        ci[pl.ds(16, 16)] = jnp.zeros((16,), jnp.int32)
        ci[pl.ds(32, 16)] = jnp.zeros((16,), jnp.int32)

        def drain(ne):
            pltpu.async_copy(tab_hbm.at[ci.at[pl.ds(0, DRAIN)]], rows, sem).wait()

            def de(e, _):
                bse = cd[pl.ds(e, 16)][0]
                if with_deg:
                    dr = bse // 16
                    adeg[pl.ds(dr, 16)] = adeg[pl.ds(dr, 16)] + onef
                for c16 in range(16):
                    v = rows[pl.ds(e, 1), pl.ds(c16 * 16, 16)].reshape(16)
                    sidx = bse + c16 * 16
                    asum[pl.ds(sidx, 16)] = asum[pl.ds(sidx, 16)] + v
                    assq[pl.ds(sidx, 16)] = assq[pl.ds(sidx, 16)] + v * v
                    amin[pl.ds(sidx, 16)] = jnp.minimum(amin[pl.ds(sidx, 16)], v)
                    amax[pl.ds(sidx, 16)] = jnp.maximum(amax[pl.ds(sidx, 16)], v)
                return 0
            lax.fori_loop(0, ne, de, 0)

        def rnd(r, _):
            rbase = b * BRANGE + r * RROWS

            def initr(g, _):
                asum[pl.ds(g * 16, 16)] = zf
                assq[pl.ds(g * 16, 16)] = zf
                amin[pl.ds(g * 16, 16)] = pinf
                amax[pl.ds(g * 16, 16)] = ninf
                return 0
            lax.fori_loop(0, RROWS * H // 16, initr, 0)
            if with_deg:
                def initd(g, _):
                    adeg[pl.ds(g * 16, 16)] = zf
                    return 0
                lax.fori_loop(0, RROWS, initd, 0)

            def sub(su, cur):
                sa = hh * 16 + su
                n = cbuf[pl.ds(sa * 16 + b, 16)][0]
                nch = (n + 255) // 256

                def chunk(c, cur):
                    pltpu.sync_copy(idx_hbm.at[b, sa, pl.ds(c * 256, 256)],
                                    ibuf.at[pl.ds(0, 256)])
                    pltpu.sync_copy(doff_hbm.at[b, sa, pl.ds(c * 256, 256)],
                                    dbuf.at[pl.ds(0, 256)])

                    def edge(e, cur):
                        dval = dbuf[pl.ds(e, 16)][0]
                        rr = dval - r * RROWS
                        ok = (rr >= 0) & (rr < RROWS)

                        @pl.when(ok)
                        def _():
                            ci[pl.ds(cur, 16)] = jnp.full(
                                (16,), ibuf[pl.ds(e, 16)][0], jnp.int32)
                            cd[pl.ds(cur, 16)] = jnp.full(
                                (16,), rr * H, jnp.int32)
                        cur2 = jnp.where(ok, cur + 1, cur)

                        @pl.when(cur2 == DRAIN)
                        def _():
                            drain(DRAIN)
                        return jnp.where(cur2 == DRAIN, 0, cur2)

                    return lax.fori_loop(0, 256, edge, cur)

                return lax.fori_loop(0, nch, chunk, cur)

            cur = lax.fori_loop(0, 16, sub, jnp.int32(0))

            @pl.when(cur > 0)
            def _():
                drain(cur)

            obase = (hh * N + rbase) * H
            dbase = (hh * N + rbase) * 16

            @pl.when(r < RND - 1)
            def _():
                sz = RROWS * H
                pltpu.sync_copy(asum.at[pl.ds(0, sz)], sum_o.at[pl.ds(obase, sz)])
                pltpu.sync_copy(amin.at[pl.ds(0, sz)], min_o.at[pl.ds(obase, sz)])
                pltpu.sync_copy(amax.at[pl.ds(0, sz)], max_o.at[pl.ds(obase, sz)])
                pltpu.sync_copy(assq.at[pl.ds(0, sz)], ssq_o.at[pl.ds(obase, sz)])
                if with_deg:
                    pltpu.sync_copy(adeg.at[pl.ds(0, RROWS * 16)],
                                    deg_o.at[pl.ds(dbase, RROWS * 16)])

            @pl.when(r == RND - 1)
            def _():
                lastr = BRANGE - (RND - 1) * RROWS
                sz = lastr * H
                pltpu.sync_copy(asum.at[pl.ds(0, sz)], sum_o.at[pl.ds(obase, sz)])
                pltpu.sync_copy(amin.at[pl.ds(0, sz)], min_o.at[pl.ds(obase, sz)])
                pltpu.sync_copy(amax.at[pl.ds(0, sz)], max_o.at[pl.ds(obase, sz)])
                pltpu.sync_copy(assq.at[pl.ds(0, sz)], ssq_o.at[pl.ds(obase, sz)])
                if with_deg:
                    pltpu.sync_copy(adeg.at[pl.ds(0, lastr * 16)],
                                    deg_o.at[pl.ds(dbase, lastr * 16)])
            return 0

        lax.fori_loop(0, RND, rnd, 0)

    return kern(table, bidx, bdoff, bcnt.reshape(NSCW * 16))


def _mm_body(a_ref, w_ref, b_ref, y_ref):
    y_ref[...] = jnp.dot(a_ref[...], w_ref[...],
                         preferred_element_type=jnp.float32) + b_ref[...]


def _matmul_bias(a, w, b, blk):
    M, K = a.shape
    F = w.shape[1]
    return pl.pallas_call(
        _mm_body,
        grid=(M // blk,),
        in_specs=[pl.BlockSpec((blk, K), lambda i: (i, 0)),
                  pl.BlockSpec((K, F), lambda i: (0, 0)),
                  pl.BlockSpec((1, F), lambda i: (0, 0))],
        out_specs=pl.BlockSpec((blk, F), lambda i: (i, 0)),
        out_shape=jax.ShapeDtypeStruct((M, F), jnp.float32),
    )(a, w, b.reshape(1, F))


def _stats4(s0, s1, q0, q1, n0, n1, x0, x1, inv, dmask):
    ssum = s0 + s1
    sssq = q0 + q1
    smin = jnp.minimum(n0, n1)
    smax = jnp.maximum(x0, x1)
    mean = ssum * inv
    std = jnp.sqrt(jax.nn.relu(sssq * inv - mean * mean) + 1e-5)
    return jnp.concatenate([
        mean,
        jnp.where(dmask, smin, 0.0),
        jnp.where(dmask, smax, 0.0),
        std,
    ], axis=1)


def _layer_body(h_ref, deg_ref,
                ss0_ref, ss1_ref, sq0_ref, sq1_ref, sn0_ref, sn1_ref,
                sx0_ref, sx1_ref,
                es0_ref, es1_ref, eq0_ref, eq1_ref, en0_ref, en1_ref,
                ex0_ref, ex1_ref,
                wdst_ref, wsrc_ref, wea_ref, c_ref,
                out_ref, psum_ref, psq_ref):
    deg = deg_ref[...]
    cntc = jnp.maximum(deg, 1.0)
    inv = 1.0 / cntc
    ld = jnp.log(jnp.maximum(deg, 1.0) + 1.0)
    amp = ld / AVG_DEG_LOG
    att = AVG_DEG_LOG / ld
    dmask = deg > 0

    s4 = _stats4(ss0_ref[...], ss1_ref[...], sq0_ref[...], sq1_ref[...],
                 sn0_ref[...], sn1_ref[...], sx0_ref[...], sx1_ref[...],
                 inv, dmask)
    e4 = _stats4(es0_ref[...], es1_ref[...], eq0_ref[...], eq1_ref[...],
                 en0_ref[...], en1_ref[...], ex0_ref[...], ex1_ref[...],
                 inv, dmask)
    hm = jnp.where(dmask, h_ref[...], 0.0)

    p = (jnp.dot(hm, wdst_ref[...], preferred_element_type=jnp.float32)
         + jnp.dot(s4, wsrc_ref[...], preferred_element_type=jnp.float32)
         + jnp.dot(e4, wea_ref[...], preferred_element_type=jnp.float32))
    c = c_ref[...]
    out = ((p[:, :H] + c[0:1, :])
           + amp * (p[:, H:2 * H] + c[1:2, :])
           + att * (p[:, 2 * H:] + c[2:3, :]))
    out_ref[...] = out
    psum_ref[...] = jnp.sum(out, axis=0, keepdims=True)[None]
    psq_ref[...] = jnp.sum(out * out, axis=0, keepdims=True)[None]


def _layer_matmul(h, deg2, ssum, smin, smax, sssq, esum, emin, emax, essq,
                  wdst, wsrc, wea, c):
    nb0 = pl.BlockSpec((NBLK, H), lambda i: (i, 0))
    nb1 = pl.BlockSpec((NBLK, H), lambda i: (i + NNB, 0))
    return pl.pallas_call(
        _layer_body,
        grid=(NNB,),
        in_specs=[nb0,
                  pl.BlockSpec((NBLK, 1), lambda i: (i, 0)),
                  nb0, nb1, nb0, nb1, nb0, nb1, nb0, nb1,
                  nb0, nb1, nb0, nb1, nb0, nb1, nb0, nb1,
                  pl.BlockSpec((H, 3 * H), lambda i: (0, 0)),
                  pl.BlockSpec((4 * H, 3 * H), lambda i: (0, 0)),
                  pl.BlockSpec((4 * H, 3 * H), lambda i: (0, 0)),
                  pl.BlockSpec((3, H), lambda i: (0, 0))],
        out_specs=[nb0,
                   pl.BlockSpec((1, 1, H), lambda i: (i, 0, 0)),
                   pl.BlockSpec((1, 1, H), lambda i: (i, 0, 0))],
        out_shape=[jax.ShapeDtypeStruct((N, H), jnp.float32),
                   jax.ShapeDtypeStruct((NNB, 1, H), jnp.float32),
                   jax.ShapeDtypeStruct((NNB, 1, H), jnp.float32)],
    )(h, deg2, ssum, ssum, sssq, sssq, smin, smin, smax, smax,
      esum, esum, essq, essq, emin, emin, emax, emax, wdst, wsrc, wea, c)


def _bn_body(out_ref, psum_ref, psq_ref, gam_ref, bet_ref, h_ref, hnew_ref):
    mu = jnp.sum(psum_ref[...], axis=0) * (1.0 / N)
    msq = jnp.sum(psq_ref[...], axis=0) * (1.0 / N)
    var = msq - mu * mu
    scale = gam_ref[...] * jax.lax.rsqrt(var + 1e-5)
    hnew_ref[...] = h_ref[...] + jax.nn.relu(
        (out_ref[...] - mu) * scale + bet_ref[...])


def _bn_apply(out, psum, psq, gamma, beta, h):
    nb = pl.BlockSpec((NBLK, H), lambda i: (i, 0))
    return pl.pallas_call(
        _bn_body,
        grid=(NNB,),
        in_specs=[nb,
                  pl.BlockSpec((NNB, 1, H), lambda i: (0, 0, 0)),
                  pl.BlockSpec((NNB, 1, H), lambda i: (0, 0, 0)),
                  pl.BlockSpec((1, H), lambda i: (0, 0)),
                  pl.BlockSpec((1, H), lambda i: (0, 0)),
                  nb],
        out_specs=nb,
        out_shape=jax.ShapeDtypeStruct((N, H), jnp.float32),
    )(out, psum, psq, gamma.reshape(1, H), beta.reshape(1, H), h)


def _head_body(h_ref, batch_ref, w1_ref, b1_ref, w2_ref, b2_ref, w3_ref, b3_ref,
               out_ref):
    gids = jax.lax.broadcasted_iota(jnp.int32, (NG, N), 0)
    onehot = (batch_ref[...] == gids).astype(jnp.float32)
    gsum = jnp.dot(onehot, h_ref[...], preferred_element_type=jnp.float32)
    gcnt = jnp.sum(onehot, axis=1, keepdims=True)
    g = gsum / jnp.maximum(gcnt, 1.0)
    z = jax.nn.relu(jnp.dot(g, w1_ref[...], preferred_element_type=jnp.float32)
                    + b1_ref[...])
    z = jax.nn.relu(jnp.dot(z, w2_ref[...], preferred_element_type=jnp.float32)
                    + b2_ref[...])
    out_ref[...] = jnp.dot(z, w3_ref[...], preferred_element_type=jnp.float32) \
        + b3_ref[...]


def _head(h, batch, fc1_W, fc1_b, fc2_W, fc2_b, fc3_W, fc3_b):
    full = lambda s: pl.BlockSpec(s, lambda: (0,) * len(s))
    return pl.pallas_call(
        _head_body,
        in_specs=[full((N, H)), full((1, N)),
                  full(fc1_W.shape), full((1, fc1_b.shape[0])),
                  full(fc2_W.shape), full((1, fc2_b.shape[0])),
                  full(fc3_W.shape), full((1, fc3_b.shape[0]))],
        out_specs=full((NG, 10)),
        out_shape=jax.ShapeDtypeStruct((NG, 10), jnp.float32),
    )(h, batch.reshape(1, N), fc1_W, fc1_b.reshape(1, -1),
      fc2_W, fc2_b.reshape(1, -1), fc3_W, fc3_b.reshape(1, -1))


def kernel(x, edge_index, batch, edge_attr, W_ne, b_ne, W_ee, b_ee, W_conv,
           b_conv, bn_gamma, bn_beta, fc1_W, fc1_b, fc2_W, fc2_b, fc3_W, fc3_b):
    src, dst = edge_index[0], edge_index[1]
    h = _matmul_bias(x, W_ne, b_ne, NBLK)
    ea = _matmul_bias(edge_attr, W_ee, b_ee, EBLK)

    beid, bsrc, bdoff, bcnt = _bucketize(src, dst)
    esum, emin, emax, essq, degf = _seg_stats(ea, beid, bdoff, bcnt, True)
    degp = degf.reshape(2, N, 16)
    deg2 = degp[0, :, :1] + degp[1, :, :1]
    esum = esum.reshape(2 * N, H)
    emin = emin.reshape(2 * N, H)
    emax = emax.reshape(2 * N, H)
    essq = essq.reshape(2 * N, H)

    wq = W_conv.reshape(NLAYERS, 3, 4, 3, H, H)
    wdst = jnp.transpose(wq[:, :, :3, 0].sum(2), (0, 2, 1, 3)).reshape(NLAYERS, H, 3 * H)
    wsrc = jnp.transpose(wq[:, :, :, 1], (0, 2, 3, 1, 4)).reshape(NLAYERS, 4 * H, 3 * H)
    wea = jnp.transpose(wq[:, :, :, 2], (0, 2, 3, 1, 4)).reshape(NLAYERS, 4 * H, 3 * H)
    cs = jnp.float32(np.sqrt(1e-5)) * wq[:, :, 3, 0].sum(axis=2)
    cs = cs.at[:, 0].add(b_conv)

    for i in range(NLAYERS):
        ssum, smin, smax, sssq = _seg_stats(h, bsrc, bdoff, bcnt, False)
        out, psum, psq = _layer_matmul(h, deg2,
                                       ssum.reshape(2 * N, H),
                                       smin.reshape(2 * N, H),
                                       smax.reshape(2 * N, H),
                                       sssq.reshape(2 * N, H),
                                       esum, emin, emax, essq,
                                       wdst[i], wsrc[i], wea[i], cs[i])
        h = _bn_apply(out, psum, psq, bn_gamma[i], bn_beta[i], h)

    return _head(h, batch, fc1_W, fc1_b, fc2_W, fc2_b, fc3_W, fc3_b)


# speedup vs baseline: 1.0122x; 1.0122x over previous
import functools
import numpy as np
import jax
import jax.numpy as jnp
from jax import lax
from jax.experimental import pallas as pl
from jax.experimental.pallas import tpu as pltpu
from jax.experimental.pallas import tpu_sc as plsc

N = 10000
E = 160000
H = 256
NG = 128
NLAYERS = 3
NBLK = 400
NNB = N // NBLK
EBLK = 1000
_DEG_HIST = np.array([0., 1000., 3000., 4000., 1500., 500.])
_b = np.arange(len(_DEG_HIST))
AVG_DEG_LOG = float((np.log(_b + 1.0) * _DEG_HIST).sum() / _DEG_HIST.sum())

NSCW = 32
PBK = 16
BRANGE = N // PBK
ECHUNK = E // NSCW
CAP = 5120
RND = 8
RROWS = 80
DRAIN = 128
NEG_INF = float(np.finfo(np.float32).min)
POS_INF = float(np.finfo(np.float32).max)


def _sc_mesh():
    return plsc.VectorSubcoreMesh(core_axis_name="c", subcore_axis_name="s")


def _bucketize(src, dst):
    @functools.partial(
        pl.kernel, mesh=_sc_mesh(),
        out_type=[jax.ShapeDtypeStruct((PBK, NSCW, CAP), jnp.int32),
                  jax.ShapeDtypeStruct((PBK, NSCW, CAP), jnp.int32),
                  jax.ShapeDtypeStruct((PBK, NSCW, CAP), jnp.int32),
                  jax.ShapeDtypeStruct((NSCW, 16), jnp.int32)],
        scratch_types=[pltpu.VMEM((CAP + 16,), jnp.int32),
                       pltpu.VMEM((CAP + 16,), jnp.int32),
                       pltpu.VMEM((CAP + 16,), jnp.int32),
                       pltpu.VMEM((CAP + 16,), jnp.int32),
                       pltpu.VMEM((CAP + 16,), jnp.int32),
                       pltpu.VMEM((16,), jnp.int32)],
    )
    def kern(src_hbm, dst_hbm, eid_out, src_out, doff_out, cnt_out,
             sbuf, dbuf, oe, os_, od, crow):
        w = lax.axis_index("s") * 2 + lax.axis_index("c")
        base = w * ECHUNK
        lanes = lax.iota(jnp.int32, 16)
        zero = jnp.zeros((16,), jnp.int32)
        pltpu.sync_copy(src_hbm.at[pl.ds(base, ECHUNK)], sbuf.at[pl.ds(0, ECHUNK)])
        pltpu.sync_copy(dst_hbm.at[pl.ds(base, ECHUNK)], dbuf.at[pl.ds(0, ECHUNK)])

        def init(g, _):
            oe[pl.ds(g * 16, 16)] = zero
            os_[pl.ds(g * 16, 16)] = zero
            return 0
        lax.fori_loop(0, (CAP + 16) // 16, init, 0)

        counts = jnp.zeros((16,), jnp.int32)
        for pb in range(PBK):
            lo = pb * BRANGE

            def edge(e, cur):
                dval = dbuf[pl.ds(e, 16)][0]
                sval = sbuf[pl.ds(e, 16)][0]
                ok = (dval >= lo) & (dval < lo + BRANGE)

                @pl.when(ok)
                def _():
                    oe[pl.ds(cur, 16)] = jnp.full((16,), base + e, jnp.int32)
                    os_[pl.ds(cur, 16)] = jnp.full((16,), sval, jnp.int32)
                    od[pl.ds(cur, 16)] = jnp.full((16,), dval - lo, jnp.int32)
                return jnp.where(ok, cur + 1, cur)

            cnt = lax.fori_loop(0, ECHUNK, edge, jnp.int32(0))
            pad = (256 - cnt % 256) % 256

            def padk(k, _):
                od[pl.ds(cnt + k, 16)] = jnp.full((16,), 10000, jnp.int32)
                return 0
            lax.fori_loop(0, pad, padk, 0)
            nch = (cnt + pad) // 256

            def wr(c, _):
                pltpu.sync_copy(oe.at[pl.ds(c * 256, 256)],
                                eid_out.at[pb, w, pl.ds(c * 256, 256)])
                pltpu.sync_copy(os_.at[pl.ds(c * 256, 256)],
                                src_out.at[pb, w, pl.ds(c * 256, 256)])
                pltpu.sync_copy(od.at[pl.ds(c * 256, 256)],
                                doff_out.at[pb, w, pl.ds(c * 256, 256)])
                return 0
            lax.fori_loop(0, nch, wr, 0)
            counts = jnp.where(lanes == pb, cnt, counts)
        crow[...] = counts
        pltpu.sync_copy(crow, cnt_out.at[w])

    return kern(src, dst)


def _seg_stats(table, bidx, bdoff, bcnt, with_deg):
    out_types = [jax.ShapeDtypeStruct((2 * N * H,), jnp.float32)] * 4
    if with_deg:
        out_types = out_types + [jax.ShapeDtypeStruct((2 * N * 16,), jnp.float32)]
    scr = [pltpu.VMEM((NSCW * 16 + 16,), jnp.int32),
           pltpu.VMEM((256 + 16,), jnp.int32),
           pltpu.VMEM((256 + 16,), jnp.int32),
           pltpu.VMEM((DRAIN + 16,), jnp.int32),
           pltpu.VMEM((DRAIN + 16,), jnp.int32),
           pltpu.VMEM((DRAIN, H), jnp.float32),
           pltpu.VMEM((RROWS * H,), jnp.float32),
           pltpu.VMEM((RROWS * H,), jnp.float32),
           pltpu.VMEM((RROWS * H,), jnp.float32),
           pltpu.VMEM((RROWS * H,), jnp.float32),
           pltpu.VMEM((RROWS * 16,), jnp.float32),
           pltpu.SemaphoreType.DMA]

    @functools.partial(pl.kernel, mesh=_sc_mesh(), out_type=out_types,
                       scratch_types=scr)
    def kern(tab_hbm, idx_hbm, doff_hbm, cnt_hbm, *rest):
        if with_deg:
            (sum_o, min_o, max_o, ssq_o, deg_o, cbuf, ibuf, dbuf, ci, cd,
             rows, asum, amin, amax, assq, adeg, sem) = rest
        else:
            deg_o = None
            (sum_o, min_o, max_o, ssq_o, cbuf, ibuf, dbuf, ci, cd,
             rows, asum, amin, amax, assq, adeg, sem) = rest
        w = lax.axis_index("s") * 2 + lax.axis_index("c")
        b = w // 2
        hh = w % 2
        zf = jnp.zeros((16,), jnp.float32)
        onef = jnp.ones((16,), jnp.float32)
        pinf = jnp.full((16,), POS_INF, jnp.float32)
        ninf = jnp.full((16,), NEG_INF, jnp.float32)
        pltpu.sync_copy(cnt_hbm.at[pl.ds(0, NSCW * 16)], cbuf.at[pl.ds(0, NSCW * 16)])
        def initci(g, _):
            ci[pl.ds(g * 16, 16)] = jnp.zeros((16,), jnp.int32)
            return 0
        lax.fori_loop(0, (DRAIN + 16) // 16, initci, 0)

        def drain(ne):
            pltpu.async_copy(tab_hbm.at[ci.at[pl.ds(0, DRAIN)]], rows, sem).wait()

            def de(e, _):
                bse = cd[pl.ds(e, 16)][0]
                if with_deg:
                    dr = bse // 16
                    adeg[pl.ds(dr, 16)] = adeg[pl.ds(dr, 16)] + onef
                for c16 in range(16):
                    v = rows[pl.ds(e, 1), pl.ds(c16 * 16, 16)].reshape(16)
                    sidx = bse + c16 * 16
                    asum[pl.ds(sidx, 16)] = asum[pl.ds(sidx, 16)] + v
                    assq[pl.ds(sidx, 16)] = assq[pl.ds(sidx, 16)] + v * v
                    amin[pl.ds(sidx, 16)] = jnp.minimum(amin[pl.ds(sidx, 16)], v)
                    amax[pl.ds(sidx, 16)] = jnp.maximum(amax[pl.ds(sidx, 16)], v)
                return 0
            lax.fori_loop(0, ne, de, 0)

        def rnd(r, _):
            rbase = b * BRANGE + r * RROWS

            def initr(g, _):
                asum[pl.ds(g * 16, 16)] = zf
                assq[pl.ds(g * 16, 16)] = zf
                amin[pl.ds(g * 16, 16)] = pinf
                amax[pl.ds(g * 16, 16)] = ninf
                return 0
            lax.fori_loop(0, RROWS * H // 16, initr, 0)
            if with_deg:
                def initd(g, _):
                    adeg[pl.ds(g * 16, 16)] = zf
                    return 0
                lax.fori_loop(0, RROWS, initd, 0)

            def sub(su, cur):
                sa = hh * 16 + su
                n = cbuf[pl.ds(sa * 16 + b, 16)][0]
                nch = (n + 255) // 256

                def chunk(c, cur):
                    pltpu.sync_copy(idx_hbm.at[b, sa, pl.ds(c * 256, 256)],
                                    ibuf.at[pl.ds(0, 256)])
                    pltpu.sync_copy(doff_hbm.at[b, sa, pl.ds(c * 256, 256)],
                                    dbuf.at[pl.ds(0, 256)])

                    def edge(e, cur):
                        dval = dbuf[pl.ds(e, 16)][0]
                        rr = dval - r * RROWS
                        ok = (rr >= 0) & (rr < RROWS)

                        @pl.when(ok)
                        def _():
                            ci[pl.ds(cur, 16)] = jnp.full(
                                (16,), ibuf[pl.ds(e, 16)][0], jnp.int32)
                            cd[pl.ds(cur, 16)] = jnp.full(
                                (16,), rr * H, jnp.int32)
                        cur2 = jnp.where(ok, cur + 1, cur)

                        @pl.when(cur2 == DRAIN)
                        def _():
                            drain(DRAIN)
                        return jnp.where(cur2 == DRAIN, 0, cur2)

                    return lax.fori_loop(0, 256, edge, cur)

                return lax.fori_loop(0, nch, chunk, cur)

            cur = lax.fori_loop(0, 16, sub, jnp.int32(0))

            @pl.when(cur > 0)
            def _():
                drain(cur)

            obase = (hh * N + rbase) * H
            dbase = (hh * N + rbase) * 16

            @pl.when(r < RND - 1)
            def _():
                sz = RROWS * H
                pltpu.sync_copy(asum.at[pl.ds(0, sz)], sum_o.at[pl.ds(obase, sz)])
                pltpu.sync_copy(amin.at[pl.ds(0, sz)], min_o.at[pl.ds(obase, sz)])
                pltpu.sync_copy(amax.at[pl.ds(0, sz)], max_o.at[pl.ds(obase, sz)])
                pltpu.sync_copy(assq.at[pl.ds(0, sz)], ssq_o.at[pl.ds(obase, sz)])
                if with_deg:
                    pltpu.sync_copy(adeg.at[pl.ds(0, RROWS * 16)],
                                    deg_o.at[pl.ds(dbase, RROWS * 16)])

            @pl.when(r == RND - 1)
            def _():
                lastr = BRANGE - (RND - 1) * RROWS
                sz = lastr * H
                pltpu.sync_copy(asum.at[pl.ds(0, sz)], sum_o.at[pl.ds(obase, sz)])
                pltpu.sync_copy(amin.at[pl.ds(0, sz)], min_o.at[pl.ds(obase, sz)])
                pltpu.sync_copy(amax.at[pl.ds(0, sz)], max_o.at[pl.ds(obase, sz)])
                pltpu.sync_copy(assq.at[pl.ds(0, sz)], ssq_o.at[pl.ds(obase, sz)])
                if with_deg:
                    pltpu.sync_copy(adeg.at[pl.ds(0, lastr * 16)],
                                    deg_o.at[pl.ds(dbase, lastr * 16)])
            return 0

        lax.fori_loop(0, RND, rnd, 0)

    return kern(table, bidx, bdoff, bcnt.reshape(NSCW * 16))


def _mm_body(a_ref, w_ref, b_ref, y_ref):
    y_ref[...] = jnp.dot(a_ref[...], w_ref[...],
                         preferred_element_type=jnp.float32) + b_ref[...]


def _matmul_bias(a, w, b, blk):
    M, K = a.shape
    F = w.shape[1]
    return pl.pallas_call(
        _mm_body,
        grid=(M // blk,),
        in_specs=[pl.BlockSpec((blk, K), lambda i: (i, 0)),
                  pl.BlockSpec((K, F), lambda i: (0, 0)),
                  pl.BlockSpec((1, F), lambda i: (0, 0))],
        out_specs=pl.BlockSpec((blk, F), lambda i: (i, 0)),
        out_shape=jax.ShapeDtypeStruct((M, F), jnp.float32),
    )(a, w, b.reshape(1, F))


def _stats4(s0, s1, q0, q1, n0, n1, x0, x1, inv, dmask):
    ssum = s0 + s1
    sssq = q0 + q1
    smin = jnp.minimum(n0, n1)
    smax = jnp.maximum(x0, x1)
    mean = ssum * inv
    std = jnp.sqrt(jax.nn.relu(sssq * inv - mean * mean) + 1e-5)
    return jnp.concatenate([
        mean,
        jnp.where(dmask, smin, 0.0),
        jnp.where(dmask, smax, 0.0),
        std,
    ], axis=1)


def _layer_body(h_ref, deg_ref,
                ss0_ref, ss1_ref, sq0_ref, sq1_ref, sn0_ref, sn1_ref,
                sx0_ref, sx1_ref,
                es0_ref, es1_ref, eq0_ref, eq1_ref, en0_ref, en1_ref,
                ex0_ref, ex1_ref,
                wdst_ref, wsrc_ref, wea_ref, c_ref,
                out_ref, psum_ref, psq_ref):
    deg = deg_ref[...]
    cntc = jnp.maximum(deg, 1.0)
    inv = 1.0 / cntc
    ld = jnp.log(jnp.maximum(deg, 1.0) + 1.0)
    amp = ld / AVG_DEG_LOG
    att = AVG_DEG_LOG / ld
    dmask = deg > 0

    s4 = _stats4(ss0_ref[...], ss1_ref[...], sq0_ref[...], sq1_ref[...],
                 sn0_ref[...], sn1_ref[...], sx0_ref[...], sx1_ref[...],
                 inv, dmask)
    e4 = _stats4(es0_ref[...], es1_ref[...], eq0_ref[...], eq1_ref[...],
                 en0_ref[...], en1_ref[...], ex0_ref[...], ex1_ref[...],
                 inv, dmask)
    hm = jnp.where(dmask, h_ref[...], 0.0)

    p = (jnp.dot(hm, wdst_ref[...], preferred_element_type=jnp.float32)
         + jnp.dot(s4, wsrc_ref[...], preferred_element_type=jnp.float32)
         + jnp.dot(e4, wea_ref[...], preferred_element_type=jnp.float32))
    c = c_ref[...]
    out = ((p[:, :H] + c[0:1, :])
           + amp * (p[:, H:2 * H] + c[1:2, :])
           + att * (p[:, 2 * H:] + c[2:3, :]))
    out_ref[...] = out
    psum_ref[...] = jnp.sum(out, axis=0, keepdims=True)[None]
    psq_ref[...] = jnp.sum(out * out, axis=0, keepdims=True)[None]


def _layer_matmul(h, deg2, ssum, smin, smax, sssq, esum, emin, emax, essq,
                  wdst, wsrc, wea, c):
    nb0 = pl.BlockSpec((NBLK, H), lambda i: (i, 0))
    nb1 = pl.BlockSpec((NBLK, H), lambda i: (i + NNB, 0))
    return pl.pallas_call(
        _layer_body,
        grid=(NNB,),
        in_specs=[nb0,
                  pl.BlockSpec((NBLK, 1), lambda i: (i, 0)),
                  nb0, nb1, nb0, nb1, nb0, nb1, nb0, nb1,
                  nb0, nb1, nb0, nb1, nb0, nb1, nb0, nb1,
                  pl.BlockSpec((H, 3 * H), lambda i: (0, 0)),
                  pl.BlockSpec((4 * H, 3 * H), lambda i: (0, 0)),
                  pl.BlockSpec((4 * H, 3 * H), lambda i: (0, 0)),
                  pl.BlockSpec((3, H), lambda i: (0, 0))],
        out_specs=[nb0,
                   pl.BlockSpec((1, 1, H), lambda i: (i, 0, 0)),
                   pl.BlockSpec((1, 1, H), lambda i: (i, 0, 0))],
        out_shape=[jax.ShapeDtypeStruct((N, H), jnp.float32),
                   jax.ShapeDtypeStruct((NNB, 1, H), jnp.float32),
                   jax.ShapeDtypeStruct((NNB, 1, H), jnp.float32)],
    )(h, deg2, ssum, ssum, sssq, sssq, smin, smin, smax, smax,
      esum, esum, essq, essq, emin, emin, emax, emax, wdst, wsrc, wea, c)


def _bn_body(out_ref, psum_ref, psq_ref, gam_ref, bet_ref, h_ref, hnew_ref):
    mu = jnp.sum(psum_ref[...], axis=0) * (1.0 / N)
    msq = jnp.sum(psq_ref[...], axis=0) * (1.0 / N)
    var = msq - mu * mu
    scale = gam_ref[...] * jax.lax.rsqrt(var + 1e-5)
    hnew_ref[...] = h_ref[...] + jax.nn.relu(
        (out_ref[...] - mu) * scale + bet_ref[...])


def _bn_apply(out, psum, psq, gamma, beta, h):
    nb = pl.BlockSpec((NBLK, H), lambda i: (i, 0))
    return pl.pallas_call(
        _bn_body,
        grid=(NNB,),
        in_specs=[nb,
                  pl.BlockSpec((NNB, 1, H), lambda i: (0, 0, 0)),
                  pl.BlockSpec((NNB, 1, H), lambda i: (0, 0, 0)),
                  pl.BlockSpec((1, H), lambda i: (0, 0)),
                  pl.BlockSpec((1, H), lambda i: (0, 0)),
                  nb],
        out_specs=nb,
        out_shape=jax.ShapeDtypeStruct((N, H), jnp.float32),
    )(out, psum, psq, gamma.reshape(1, H), beta.reshape(1, H), h)


def _head_body(h_ref, batch_ref, w1_ref, b1_ref, w2_ref, b2_ref, w3_ref, b3_ref,
               out_ref):
    gids = jax.lax.broadcasted_iota(jnp.int32, (NG, N), 0)
    onehot = (batch_ref[...] == gids).astype(jnp.float32)
    gsum = jnp.dot(onehot, h_ref[...], preferred_element_type=jnp.float32)
    gcnt = jnp.sum(onehot, axis=1, keepdims=True)
    g = gsum / jnp.maximum(gcnt, 1.0)
    z = jax.nn.relu(jnp.dot(g, w1_ref[...], preferred_element_type=jnp.float32)
                    + b1_ref[...])
    z = jax.nn.relu(jnp.dot(z, w2_ref[...], preferred_element_type=jnp.float32)
                    + b2_ref[...])
    out_ref[...] = jnp.dot(z, w3_ref[...], preferred_element_type=jnp.float32) \
        + b3_ref[...]


def _head(h, batch, fc1_W, fc1_b, fc2_W, fc2_b, fc3_W, fc3_b):
    full = lambda s: pl.BlockSpec(s, lambda: (0,) * len(s))
    return pl.pallas_call(
        _head_body,
        in_specs=[full((N, H)), full((1, N)),
                  full(fc1_W.shape), full((1, fc1_b.shape[0])),
                  full(fc2_W.shape), full((1, fc2_b.shape[0])),
                  full(fc3_W.shape), full((1, fc3_b.shape[0]))],
        out_specs=full((NG, 10)),
        out_shape=jax.ShapeDtypeStruct((NG, 10), jnp.float32),
    )(h, batch.reshape(1, N), fc1_W, fc1_b.reshape(1, -1),
      fc2_W, fc2_b.reshape(1, -1), fc3_W, fc3_b.reshape(1, -1))


def kernel(x, edge_index, batch, edge_attr, W_ne, b_ne, W_ee, b_ee, W_conv,
           b_conv, bn_gamma, bn_beta, fc1_W, fc1_b, fc2_W, fc2_b, fc3_W, fc3_b):
    src, dst = edge_index[0], edge_index[1]
    h = _matmul_bias(x, W_ne, b_ne, NBLK)
    ea = _matmul_bias(edge_attr, W_ee, b_ee, EBLK)

    beid, bsrc, bdoff, bcnt = _bucketize(src, dst)
    esum, emin, emax, essq, degf = _seg_stats(ea, beid, bdoff, bcnt, True)
    degp = degf.reshape(2, N, 16)
    deg2 = degp[0, :, :1] + degp[1, :, :1]
    esum = esum.reshape(2 * N, H)
    emin = emin.reshape(2 * N, H)
    emax = emax.reshape(2 * N, H)
    essq = essq.reshape(2 * N, H)

    wq = W_conv.reshape(NLAYERS, 3, 4, 3, H, H)
    wdst = jnp.transpose(wq[:, :, :3, 0].sum(2), (0, 2, 1, 3)).reshape(NLAYERS, H, 3 * H)
    wsrc = jnp.transpose(wq[:, :, :, 1], (0, 2, 3, 1, 4)).reshape(NLAYERS, 4 * H, 3 * H)
    wea = jnp.transpose(wq[:, :, :, 2], (0, 2, 3, 1, 4)).reshape(NLAYERS, 4 * H, 3 * H)
    cs = jnp.float32(np.sqrt(1e-5)) * wq[:, :, 3, 0].sum(axis=2)
    cs = cs.at[:, 0].add(b_conv)

    for i in range(NLAYERS):
        ssum, smin, smax, sssq = _seg_stats(h, bsrc, bdoff, bcnt, False)
        out, psum, psq = _layer_matmul(h, deg2,
                                       ssum.reshape(2 * N, H),
                                       smin.reshape(2 * N, H),
                                       smax.reshape(2 * N, H),
                                       sssq.reshape(2 * N, H),
                                       esum, emin, emax, essq,
                                       wdst[i], wsrc[i], wea[i], cs[i])
        h = _bn_apply(out, psum, psq, bn_gamma[i], bn_beta[i], h)

    return _head(h, batch, fc1_W, fc1_b, fc2_W, fc2_b, fc3_W, fc3_b)


# speedup vs baseline: 1.3009x; 1.2852x over previous
import functools
import numpy as np
import jax
import jax.numpy as jnp
from jax import lax
from jax.experimental import pallas as pl
from jax.experimental.pallas import tpu as pltpu
from jax.experimental.pallas import tpu_sc as plsc

N = 10000
E = 160000
H = 256
NG = 128
NLAYERS = 3
NBLK = 400
NNB = N // NBLK
EBLK = 1000
_DEG_HIST = np.array([0., 1000., 3000., 4000., 1500., 500.])
_b = np.arange(len(_DEG_HIST))
AVG_DEG_LOG = float((np.log(_b + 1.0) * _DEG_HIST).sum() / _DEG_HIST.sum())

NSCW = 32
PBK = 16
BRANGE = N // PBK
ECHUNK = E // NSCW
CAP = 5120
RND = 8
RROWS = 80
DRAIN = 128
CHKB = 1024
NEG_INF = float(np.finfo(np.float32).min)
POS_INF = float(np.finfo(np.float32).max)


def _sc_mesh():
    return plsc.VectorSubcoreMesh(core_axis_name="c", subcore_axis_name="s")


def _bucketize(src, dst):
    @functools.partial(
        pl.kernel, mesh=_sc_mesh(),
        out_type=[jax.ShapeDtypeStruct((PBK, NSCW, CAP), jnp.int32),
                  jax.ShapeDtypeStruct((PBK, NSCW, CAP), jnp.int32),
                  jax.ShapeDtypeStruct((PBK, NSCW, CAP), jnp.int32),
                  jax.ShapeDtypeStruct((NSCW, 16), jnp.int32)],
        scratch_types=[pltpu.VMEM((CAP + 16,), jnp.int32),
                       pltpu.VMEM((CAP + 16,), jnp.int32),
                       pltpu.VMEM((CAP + 16,), jnp.int32),
                       pltpu.VMEM((CAP + 16,), jnp.int32),
                       pltpu.VMEM((CAP + 16,), jnp.int32),
                       pltpu.VMEM((16,), jnp.int32)],
    )
    def kern(src_hbm, dst_hbm, eid_out, src_out, doff_out, cnt_out,
             sbuf, dbuf, oe, os_, od, crow):
        w = lax.axis_index("s") * 2 + lax.axis_index("c")
        base = w * ECHUNK
        lanes = lax.iota(jnp.int32, 16)
        zero = jnp.zeros((16,), jnp.int32)
        pltpu.sync_copy(src_hbm.at[pl.ds(base, ECHUNK)], sbuf.at[pl.ds(0, ECHUNK)])
        pltpu.sync_copy(dst_hbm.at[pl.ds(base, ECHUNK)], dbuf.at[pl.ds(0, ECHUNK)])

        def init(g, _):
            oe[pl.ds(g * 16, 16)] = zero
            os_[pl.ds(g * 16, 16)] = zero
            return 0
        lax.fori_loop(0, (CAP + 16) // 16, init, 0)

        counts = jnp.zeros((16,), jnp.int32)
        for pb in range(PBK):
            lo = pb * BRANGE

            def edge(e, cur):
                dval = dbuf[pl.ds(e, 16)][0]
                sval = sbuf[pl.ds(e, 16)][0]
                ok = (dval >= lo) & (dval < lo + BRANGE)

                @pl.when(ok)
                def _():
                    oe[pl.ds(cur, 16)] = jnp.full((16,), base + e, jnp.int32)
                    os_[pl.ds(cur, 16)] = jnp.full((16,), sval, jnp.int32)
                    od[pl.ds(cur, 16)] = jnp.full((16,), dval - lo, jnp.int32)
                return jnp.where(ok, cur + 1, cur)

            cnt = lax.fori_loop(0, ECHUNK, edge, jnp.int32(0))
            pad = (256 - cnt % 256) % 256

            def padk(k, _):
                od[pl.ds(cnt + k, 16)] = jnp.full((16,), 10000, jnp.int32)
                return 0
            lax.fori_loop(0, pad, padk, 0)
            nch = (cnt + pad) // 256

            def wr(c, _):
                pltpu.sync_copy(oe.at[pl.ds(c * 256, 256)],
                                eid_out.at[pb, w, pl.ds(c * 256, 256)])
                pltpu.sync_copy(os_.at[pl.ds(c * 256, 256)],
                                src_out.at[pb, w, pl.ds(c * 256, 256)])
                pltpu.sync_copy(od.at[pl.ds(c * 256, 256)],
                                doff_out.at[pb, w, pl.ds(c * 256, 256)])
                return 0
            lax.fori_loop(0, nch, wr, 0)
            counts = jnp.where(lanes == pb, cnt, counts)
        crow[...] = counts
        pltpu.sync_copy(crow, cnt_out.at[w])

    return kern(src, dst)


def _seg_stats(table, bidx, bdoff, bcnt, with_deg):
    out_types = [jax.ShapeDtypeStruct((2 * N * H,), jnp.float32)] * 4
    if with_deg:
        out_types = out_types + [jax.ShapeDtypeStruct((2 * N * 16,), jnp.float32)]
    scr = [pltpu.VMEM((NSCW * 16 + 16,), jnp.int32),
           pltpu.VMEM((CHKB + 16,), jnp.int32),
           pltpu.VMEM((CHKB + 16,), jnp.int32),
           pltpu.VMEM((DRAIN + 16,), jnp.int32),
           pltpu.VMEM((DRAIN + 16,), jnp.int32),
           pltpu.VMEM((DRAIN, H), jnp.float32),
           pltpu.VMEM((RROWS * H,), jnp.float32),
           pltpu.VMEM((RROWS * H,), jnp.float32),
           pltpu.VMEM((RROWS * H,), jnp.float32),
           pltpu.VMEM((RROWS * H,), jnp.float32),
           pltpu.VMEM((RROWS * 16,), jnp.float32),
           pltpu.SemaphoreType.DMA]

    @functools.partial(pl.kernel, mesh=_sc_mesh(), out_type=out_types,
                       scratch_types=scr)
    def kern(tab_hbm, idx_hbm, doff_hbm, cnt_hbm, *rest):
        if with_deg:
            (sum_o, min_o, max_o, ssq_o, deg_o, cbuf, ibuf, dbuf, ci, cd,
             rows, asum, amin, amax, assq, adeg, sem) = rest
        else:
            deg_o = None
            (sum_o, min_o, max_o, ssq_o, cbuf, ibuf, dbuf, ci, cd,
             rows, asum, amin, amax, assq, adeg, sem) = rest
        w = lax.axis_index("s") * 2 + lax.axis_index("c")
        b = w // 2
        hh = w % 2
        zf = jnp.zeros((16,), jnp.float32)
        onef = jnp.ones((16,), jnp.float32)
        pinf = jnp.full((16,), POS_INF, jnp.float32)
        ninf = jnp.full((16,), NEG_INF, jnp.float32)
        pltpu.sync_copy(cnt_hbm.at[pl.ds(0, NSCW * 16)], cbuf.at[pl.ds(0, NSCW * 16)])
        def initci(g, _):
            ci[pl.ds(g * 16, 16)] = jnp.zeros((16,), jnp.int32)
            return 0
        lax.fori_loop(0, (DRAIN + 16) // 16, initci, 0)

        def drain(ne):
            pltpu.async_copy(tab_hbm.at[ci.at[pl.ds(0, DRAIN)]], rows, sem).wait()

            def de(e, _):
                bse = cd[pl.ds(e, 16)][0]
                if with_deg:
                    dr = bse // 16
                    adeg[pl.ds(dr, 16)] = adeg[pl.ds(dr, 16)] + onef
                for c16 in range(16):
                    v = rows[pl.ds(e, 1), pl.ds(c16 * 16, 16)].reshape(16)
                    sidx = bse + c16 * 16
                    asum[pl.ds(sidx, 16)] = asum[pl.ds(sidx, 16)] + v
                    assq[pl.ds(sidx, 16)] = assq[pl.ds(sidx, 16)] + v * v
                    amin[pl.ds(sidx, 16)] = jnp.minimum(amin[pl.ds(sidx, 16)], v)
                    amax[pl.ds(sidx, 16)] = jnp.maximum(amax[pl.ds(sidx, 16)], v)
                return 0
            lax.fori_loop(0, ne, de, 0)

        def rnd(r, _):
            rbase = b * BRANGE + r * RROWS

            def initr(g, _):
                asum[pl.ds(g * 16, 16)] = zf
                assq[pl.ds(g * 16, 16)] = zf
                amin[pl.ds(g * 16, 16)] = pinf
                amax[pl.ds(g * 16, 16)] = ninf
                return 0
            lax.fori_loop(0, RROWS * H // 16, initr, 0)
            if with_deg:
                def initd(g, _):
                    adeg[pl.ds(g * 16, 16)] = zf
                    return 0
                lax.fori_loop(0, RROWS, initd, 0)

            def sub(su, cur):
                sa = hh * 16 + su
                n = cbuf[pl.ds(sa * 16 + b, 16)][0]
                nch = (n + CHKB - 1) // CHKB

                def chunk(c, cur):
                    nc = jnp.minimum(n - c * CHKB, CHKB)
                    pltpu.sync_copy(idx_hbm.at[b, sa, pl.ds(c * CHKB, CHKB)],
                                    ibuf.at[pl.ds(0, CHKB)])
                    pltpu.sync_copy(doff_hbm.at[b, sa, pl.ds(c * CHKB, CHKB)],
                                    dbuf.at[pl.ds(0, CHKB)])

                    def edge(e, cur):
                        dval = dbuf[pl.ds(e, 16)][0]
                        rr = dval - r * RROWS
                        ok = (rr >= 0) & (rr < RROWS)

                        @pl.when(ok)
                        def _():
                            ci[pl.ds(cur, 16)] = jnp.full(
                                (16,), ibuf[pl.ds(e, 16)][0], jnp.int32)
                            cd[pl.ds(cur, 16)] = jnp.full(
                                (16,), rr * H, jnp.int32)
                        cur2 = jnp.where(ok, cur + 1, cur)

                        @pl.when(cur2 == DRAIN)
                        def _():
                            drain(DRAIN)
                        return jnp.where(cur2 == DRAIN, 0, cur2)

                    return lax.fori_loop(0, nc, edge, cur)

                return lax.fori_loop(0, nch, chunk, cur)

            cur = lax.fori_loop(0, 16, sub, jnp.int32(0))

            @pl.when(cur > 0)
            def _():
                drain(cur)

            obase = (hh * N + rbase) * H
            dbase = (hh * N + rbase) * 16

            @pl.when(r < RND - 1)
            def _():
                sz = RROWS * H
                pltpu.sync_copy(asum.at[pl.ds(0, sz)], sum_o.at[pl.ds(obase, sz)])
                pltpu.sync_copy(amin.at[pl.ds(0, sz)], min_o.at[pl.ds(obase, sz)])
                pltpu.sync_copy(amax.at[pl.ds(0, sz)], max_o.at[pl.ds(obase, sz)])
                pltpu.sync_copy(assq.at[pl.ds(0, sz)], ssq_o.at[pl.ds(obase, sz)])
                if with_deg:
                    pltpu.sync_copy(adeg.at[pl.ds(0, RROWS * 16)],
                                    deg_o.at[pl.ds(dbase, RROWS * 16)])

            @pl.when(r == RND - 1)
            def _():
                lastr = BRANGE - (RND - 1) * RROWS
                sz = lastr * H
                pltpu.sync_copy(asum.at[pl.ds(0, sz)], sum_o.at[pl.ds(obase, sz)])
                pltpu.sync_copy(amin.at[pl.ds(0, sz)], min_o.at[pl.ds(obase, sz)])
                pltpu.sync_copy(amax.at[pl.ds(0, sz)], max_o.at[pl.ds(obase, sz)])
                pltpu.sync_copy(assq.at[pl.ds(0, sz)], ssq_o.at[pl.ds(obase, sz)])
                if with_deg:
                    pltpu.sync_copy(adeg.at[pl.ds(0, lastr * 16)],
                                    deg_o.at[pl.ds(dbase, lastr * 16)])
            return 0

        lax.fori_loop(0, RND, rnd, 0)

    return kern(table, bidx, bdoff, bcnt.reshape(NSCW * 16))


def _mm_body(a_ref, w_ref, b_ref, y_ref):
    y_ref[...] = jnp.dot(a_ref[...], w_ref[...],
                         preferred_element_type=jnp.float32) + b_ref[...]


def _matmul_bias(a, w, b, blk):
    M, K = a.shape
    F = w.shape[1]
    return pl.pallas_call(
        _mm_body,
        grid=(M // blk,),
        in_specs=[pl.BlockSpec((blk, K), lambda i: (i, 0)),
                  pl.BlockSpec((K, F), lambda i: (0, 0)),
                  pl.BlockSpec((1, F), lambda i: (0, 0))],
        out_specs=pl.BlockSpec((blk, F), lambda i: (i, 0)),
        out_shape=jax.ShapeDtypeStruct((M, F), jnp.float32),
    )(a, w, b.reshape(1, F))


def _stats4(s0, s1, q0, q1, n0, n1, x0, x1, inv, dmask):
    ssum = s0 + s1
    sssq = q0 + q1
    smin = jnp.minimum(n0, n1)
    smax = jnp.maximum(x0, x1)
    mean = ssum * inv
    std = jnp.sqrt(jax.nn.relu(sssq * inv - mean * mean) + 1e-5)
    return jnp.concatenate([
        mean,
        jnp.where(dmask, smin, 0.0),
        jnp.where(dmask, smax, 0.0),
        std,
    ], axis=1)


def _layer_body(h_ref, deg_ref,
                ss0_ref, ss1_ref, sq0_ref, sq1_ref, sn0_ref, sn1_ref,
                sx0_ref, sx1_ref,
                es0_ref, es1_ref, eq0_ref, eq1_ref, en0_ref, en1_ref,
                ex0_ref, ex1_ref,
                wdst_ref, wsrc_ref, wea_ref, c_ref,
                out_ref, psum_ref, psq_ref):
    deg = deg_ref[...]
    cntc = jnp.maximum(deg, 1.0)
    inv = 1.0 / cntc
    ld = jnp.log(jnp.maximum(deg, 1.0) + 1.0)
    amp = ld / AVG_DEG_LOG
    att = AVG_DEG_LOG / ld
    dmask = deg > 0

    s4 = _stats4(ss0_ref[...], ss1_ref[...], sq0_ref[...], sq1_ref[...],
                 sn0_ref[...], sn1_ref[...], sx0_ref[...], sx1_ref[...],
                 inv, dmask)
    e4 = _stats4(es0_ref[...], es1_ref[...], eq0_ref[...], eq1_ref[...],
                 en0_ref[...], en1_ref[...], ex0_ref[...], ex1_ref[...],
                 inv, dmask)
    hm = jnp.where(dmask, h_ref[...], 0.0)

    p = (jnp.dot(hm, wdst_ref[...], preferred_element_type=jnp.float32)
         + jnp.dot(s4, wsrc_ref[...], preferred_element_type=jnp.float32)
         + jnp.dot(e4, wea_ref[...], preferred_element_type=jnp.float32))
    c = c_ref[...]
    out = ((p[:, :H] + c[0:1, :])
           + amp * (p[:, H:2 * H] + c[1:2, :])
           + att * (p[:, 2 * H:] + c[2:3, :]))
    out_ref[...] = out
    psum_ref[...] = jnp.sum(out, axis=0, keepdims=True)[None]
    psq_ref[...] = jnp.sum(out * out, axis=0, keepdims=True)[None]


def _layer_matmul(h, deg2, ssum, smin, smax, sssq, esum, emin, emax, essq,
                  wdst, wsrc, wea, c):
    nb0 = pl.BlockSpec((NBLK, H), lambda i: (i, 0))
    nb1 = pl.BlockSpec((NBLK, H), lambda i: (i + NNB, 0))
    return pl.pallas_call(
        _layer_body,
        grid=(NNB,),
        in_specs=[nb0,
                  pl.BlockSpec((NBLK, 1), lambda i: (i, 0)),
                  nb0, nb1, nb0, nb1, nb0, nb1, nb0, nb1,
                  nb0, nb1, nb0, nb1, nb0, nb1, nb0, nb1,
                  pl.BlockSpec((H, 3 * H), lambda i: (0, 0)),
                  pl.BlockSpec((4 * H, 3 * H), lambda i: (0, 0)),
                  pl.BlockSpec((4 * H, 3 * H), lambda i: (0, 0)),
                  pl.BlockSpec((3, H), lambda i: (0, 0))],
        out_specs=[nb0,
                   pl.BlockSpec((1, 1, H), lambda i: (i, 0, 0)),
                   pl.BlockSpec((1, 1, H), lambda i: (i, 0, 0))],
        out_shape=[jax.ShapeDtypeStruct((N, H), jnp.float32),
                   jax.ShapeDtypeStruct((NNB, 1, H), jnp.float32),
                   jax.ShapeDtypeStruct((NNB, 1, H), jnp.float32)],
    )(h, deg2, ssum, ssum, sssq, sssq, smin, smin, smax, smax,
      esum, esum, essq, essq, emin, emin, emax, emax, wdst, wsrc, wea, c)


def _bn_body(out_ref, psum_ref, psq_ref, gam_ref, bet_ref, h_ref, hnew_ref):
    mu = jnp.sum(psum_ref[...], axis=0) * (1.0 / N)
    msq = jnp.sum(psq_ref[...], axis=0) * (1.0 / N)
    var = msq - mu * mu
    scale = gam_ref[...] * jax.lax.rsqrt(var + 1e-5)
    hnew_ref[...] = h_ref[...] + jax.nn.relu(
        (out_ref[...] - mu) * scale + bet_ref[...])


def _bn_apply(out, psum, psq, gamma, beta, h):
    nb = pl.BlockSpec((NBLK, H), lambda i: (i, 0))
    return pl.pallas_call(
        _bn_body,
        grid=(NNB,),
        in_specs=[nb,
                  pl.BlockSpec((NNB, 1, H), lambda i: (0, 0, 0)),
                  pl.BlockSpec((NNB, 1, H), lambda i: (0, 0, 0)),
                  pl.BlockSpec((1, H), lambda i: (0, 0)),
                  pl.BlockSpec((1, H), lambda i: (0, 0)),
                  nb],
        out_specs=nb,
        out_shape=jax.ShapeDtypeStruct((N, H), jnp.float32),
    )(out, psum, psq, gamma.reshape(1, H), beta.reshape(1, H), h)


def _head_body(h_ref, batch_ref, w1_ref, b1_ref, w2_ref, b2_ref, w3_ref, b3_ref,
               out_ref):
    gids = jax.lax.broadcasted_iota(jnp.int32, (NG, N), 0)
    onehot = (batch_ref[...] == gids).astype(jnp.float32)
    gsum = jnp.dot(onehot, h_ref[...], preferred_element_type=jnp.float32)
    gcnt = jnp.sum(onehot, axis=1, keepdims=True)
    g = gsum / jnp.maximum(gcnt, 1.0)
    z = jax.nn.relu(jnp.dot(g, w1_ref[...], preferred_element_type=jnp.float32)
                    + b1_ref[...])
    z = jax.nn.relu(jnp.dot(z, w2_ref[...], preferred_element_type=jnp.float32)
                    + b2_ref[...])
    out_ref[...] = jnp.dot(z, w3_ref[...], preferred_element_type=jnp.float32) \
        + b3_ref[...]


def _head(h, batch, fc1_W, fc1_b, fc2_W, fc2_b, fc3_W, fc3_b):
    full = lambda s: pl.BlockSpec(s, lambda: (0,) * len(s))
    return pl.pallas_call(
        _head_body,
        in_specs=[full((N, H)), full((1, N)),
                  full(fc1_W.shape), full((1, fc1_b.shape[0])),
                  full(fc2_W.shape), full((1, fc2_b.shape[0])),
                  full(fc3_W.shape), full((1, fc3_b.shape[0]))],
        out_specs=full((NG, 10)),
        out_shape=jax.ShapeDtypeStruct((NG, 10), jnp.float32),
    )(h, batch.reshape(1, N), fc1_W, fc1_b.reshape(1, -1),
      fc2_W, fc2_b.reshape(1, -1), fc3_W, fc3_b.reshape(1, -1))


def kernel(x, edge_index, batch, edge_attr, W_ne, b_ne, W_ee, b_ee, W_conv,
           b_conv, bn_gamma, bn_beta, fc1_W, fc1_b, fc2_W, fc2_b, fc3_W, fc3_b):
    src, dst = edge_index[0], edge_index[1]
    h = _matmul_bias(x, W_ne, b_ne, NBLK)
    ea = _matmul_bias(edge_attr, W_ee, b_ee, EBLK)

    beid, bsrc, bdoff, bcnt = _bucketize(src, dst)
    esum, emin, emax, essq, degf = _seg_stats(ea, beid, bdoff, bcnt, True)
    degp = degf.reshape(2, N, 16)
    deg2 = degp[0, :, :1] + degp[1, :, :1]
    esum = esum.reshape(2 * N, H)
    emin = emin.reshape(2 * N, H)
    emax = emax.reshape(2 * N, H)
    essq = essq.reshape(2 * N, H)

    wq = W_conv.reshape(NLAYERS, 3, 4, 3, H, H)
    wdst = jnp.transpose(wq[:, :, :3, 0].sum(2), (0, 2, 1, 3)).reshape(NLAYERS, H, 3 * H)
    wsrc = jnp.transpose(wq[:, :, :, 1], (0, 2, 3, 1, 4)).reshape(NLAYERS, 4 * H, 3 * H)
    wea = jnp.transpose(wq[:, :, :, 2], (0, 2, 3, 1, 4)).reshape(NLAYERS, 4 * H, 3 * H)
    cs = jnp.float32(np.sqrt(1e-5)) * wq[:, :, 3, 0].sum(axis=2)
    cs = cs.at[:, 0].add(b_conv)

    for i in range(NLAYERS):
        ssum, smin, smax, sssq = _seg_stats(h, bsrc, bdoff, bcnt, False)
        out, psum, psq = _layer_matmul(h, deg2,
                                       ssum.reshape(2 * N, H),
                                       smin.reshape(2 * N, H),
                                       smax.reshape(2 * N, H),
                                       sssq.reshape(2 * N, H),
                                       esum, emin, emax, essq,
                                       wdst[i], wsrc[i], wea[i], cs[i])
        h = _bn_apply(out, psum, psq, bn_gamma[i], bn_beta[i], h)

    return _head(h, batch, fc1_W, fc1_b, fc2_W, fc2_b, fc3_W, fc3_b)


# speedup vs baseline: 1.4373x; 1.1048x over previous
import functools
import numpy as np
import jax
import jax.numpy as jnp
from jax import lax
from jax.experimental import pallas as pl
from jax.experimental.pallas import tpu as pltpu
from jax.experimental.pallas import tpu_sc as plsc

N = 10000
E = 160000
H = 256
NG = 128
NLAYERS = 3
NBLK = 400
NNB = N // NBLK
EBLK = 1000
_DEG_HIST = np.array([0., 1000., 3000., 4000., 1500., 500.])
_b = np.arange(len(_DEG_HIST))
AVG_DEG_LOG = float((np.log(_b + 1.0) * _DEG_HIST).sum() / _DEG_HIST.sum())

NSCW = 32
PBK = 16
BRANGE = N // PBK
ECHUNK = E // NSCW
CAP = 5120
RND = 8
RROWS = 80
DRAIN = 128
CHKB = 1024
NEG_INF = float(np.finfo(np.float32).min)
POS_INF = float(np.finfo(np.float32).max)


def _sc_mesh():
    return plsc.VectorSubcoreMesh(core_axis_name="c", subcore_axis_name="s")


def _bucketize(src, dst):
    @functools.partial(
        pl.kernel, mesh=_sc_mesh(),
        out_type=[jax.ShapeDtypeStruct((PBK, NSCW, CAP), jnp.int32),
                  jax.ShapeDtypeStruct((PBK, NSCW, CAP), jnp.int32),
                  jax.ShapeDtypeStruct((PBK, NSCW, CAP), jnp.int32),
                  jax.ShapeDtypeStruct((NSCW, 16), jnp.int32)],
        scratch_types=[pltpu.VMEM((CAP + 16,), jnp.int32),
                       pltpu.VMEM((CAP + 16,), jnp.int32),
                       pltpu.VMEM((CAP + 16,), jnp.int32),
                       pltpu.VMEM((CAP + 16,), jnp.int32),
                       pltpu.VMEM((CAP + 16,), jnp.int32),
                       pltpu.VMEM((16,), jnp.int32)],
    )
    def kern(src_hbm, dst_hbm, eid_out, src_out, doff_out, cnt_out,
             sbuf, dbuf, oe, os_, od, crow):
        w = lax.axis_index("s") * 2 + lax.axis_index("c")
        base = w * ECHUNK
        lanes = lax.iota(jnp.int32, 16)
        zero = jnp.zeros((16,), jnp.int32)
        pltpu.sync_copy(src_hbm.at[pl.ds(base, ECHUNK)], sbuf.at[pl.ds(0, ECHUNK)])
        pltpu.sync_copy(dst_hbm.at[pl.ds(base, ECHUNK)], dbuf.at[pl.ds(0, ECHUNK)])

        def init(g, _):
            oe[pl.ds(g * 16, 16)] = zero
            os_[pl.ds(g * 16, 16)] = zero
            return 0
        lax.fori_loop(0, (CAP + 16) // 16, init, 0)

        counts = jnp.zeros((16,), jnp.int32)
        for pb in range(PBK):
            lo = pb * BRANGE

            def edge(e, cur):
                dval = dbuf[pl.ds(e, 16)][0]
                sval = sbuf[pl.ds(e, 16)][0]
                ok = (dval >= lo) & (dval < lo + BRANGE)

                @pl.when(ok)
                def _():
                    oe[pl.ds(cur, 16)] = jnp.full((16,), base + e, jnp.int32)
                    os_[pl.ds(cur, 16)] = jnp.full((16,), sval, jnp.int32)
                    od[pl.ds(cur, 16)] = jnp.full((16,), dval - lo, jnp.int32)
                return jnp.where(ok, cur + 1, cur)

            cnt = lax.fori_loop(0, ECHUNK, edge, jnp.int32(0))
            pad = (256 - cnt % 256) % 256

            def padk(k, _):
                od[pl.ds(cnt + k, 16)] = jnp.full((16,), 10000, jnp.int32)
                return 0
            lax.fori_loop(0, pad, padk, 0)
            nch = (cnt + pad) // 256

            def wr(c, _):
                pltpu.sync_copy(oe.at[pl.ds(c * 256, 256)],
                                eid_out.at[pb, w, pl.ds(c * 256, 256)])
                pltpu.sync_copy(os_.at[pl.ds(c * 256, 256)],
                                src_out.at[pb, w, pl.ds(c * 256, 256)])
                pltpu.sync_copy(od.at[pl.ds(c * 256, 256)],
                                doff_out.at[pb, w, pl.ds(c * 256, 256)])
                return 0
            lax.fori_loop(0, nch, wr, 0)
            counts = jnp.where(lanes == pb, cnt, counts)
        crow[...] = counts
        pltpu.sync_copy(crow, cnt_out.at[w])

    return kern(src, dst)


def _seg_stats(table, bidx, bdoff, bcnt, with_deg):
    out_types = [jax.ShapeDtypeStruct((2 * N * H,), jnp.float32)] * 4
    if with_deg:
        out_types = out_types + [jax.ShapeDtypeStruct((2 * N * 16,), jnp.float32)]
    scr = [pltpu.VMEM((NSCW * 16 + 16,), jnp.int32),
           pltpu.VMEM((CHKB + 16,), jnp.int32),
           pltpu.VMEM((CHKB + 16,), jnp.int32),
           pltpu.VMEM((DRAIN + 16,), jnp.int32),
           pltpu.VMEM((DRAIN + 16,), jnp.int32),
           pltpu.VMEM((DRAIN, H), jnp.float32),
           pltpu.VMEM((RROWS * H,), jnp.float32),
           pltpu.VMEM((RROWS * H,), jnp.float32),
           pltpu.VMEM((RROWS * H,), jnp.float32),
           pltpu.VMEM((RROWS * H,), jnp.float32),
           pltpu.VMEM((RROWS * 16,), jnp.float32),
           pltpu.SemaphoreType.DMA]

    @functools.partial(pl.kernel, mesh=_sc_mesh(), out_type=out_types,
                       scratch_types=scr)
    def kern(tab_hbm, idx_hbm, doff_hbm, cnt_hbm, *rest):
        if with_deg:
            (sum_o, min_o, max_o, ssq_o, deg_o, cbuf, ibuf, dbuf, ci, cd,
             rows, asum, amin, amax, assq, adeg, sem) = rest
        else:
            deg_o = None
            (sum_o, min_o, max_o, ssq_o, cbuf, ibuf, dbuf, ci, cd,
             rows, asum, amin, amax, assq, adeg, sem) = rest
        w = lax.axis_index("s") * 2 + lax.axis_index("c")
        b = w // 2
        hh = w % 2
        zf = jnp.zeros((16,), jnp.float32)
        onef = jnp.ones((16,), jnp.float32)
        pinf = jnp.full((16,), POS_INF, jnp.float32)
        ninf = jnp.full((16,), NEG_INF, jnp.float32)
        pltpu.sync_copy(cnt_hbm.at[pl.ds(0, NSCW * 16)], cbuf.at[pl.ds(0, NSCW * 16)])
        def initci(g, _):
            ci[pl.ds(g * 16, 16)] = jnp.zeros((16,), jnp.int32)
            return 0
        lax.fori_loop(0, (DRAIN + 16) // 16, initci, 0)

        def drain(ne):
            pltpu.async_copy(tab_hbm.at[ci.at[pl.ds(0, DRAIN)]], rows, sem).wait()

            def de(e, _):
                bse = cd[pl.ds(e, 16)][0]
                if with_deg:
                    dr = bse // 16
                    adeg[pl.ds(dr, 16)] = adeg[pl.ds(dr, 16)] + onef
                for c16 in range(16):
                    v = rows[pl.ds(e, 1), pl.ds(c16 * 16, 16)].reshape(16)
                    sidx = bse + c16 * 16
                    asum[pl.ds(sidx, 16)] = asum[pl.ds(sidx, 16)] + v
                    assq[pl.ds(sidx, 16)] = assq[pl.ds(sidx, 16)] + v * v
                    amin[pl.ds(sidx, 16)] = jnp.minimum(amin[pl.ds(sidx, 16)], v)
                    amax[pl.ds(sidx, 16)] = jnp.maximum(amax[pl.ds(sidx, 16)], v)
                return 0
            lax.fori_loop(0, ne, de, 0)

        def rnd(r, _):
            rbase = b * BRANGE + r * RROWS

            def initr(g, _):
                for u in range(8):
                    asum[pl.ds(g * 128 + u * 16, 16)] = zf
                    assq[pl.ds(g * 128 + u * 16, 16)] = zf
                    amin[pl.ds(g * 128 + u * 16, 16)] = pinf
                    amax[pl.ds(g * 128 + u * 16, 16)] = ninf
                return 0
            lax.fori_loop(0, RROWS * H // 128, initr, 0)
            if with_deg:
                def initd(g, _):
                    adeg[pl.ds(g * 16, 16)] = zf
                    return 0
                lax.fori_loop(0, RROWS, initd, 0)

            def sub(su, cur):
                sa = hh * 16 + su
                n = cbuf[pl.ds(sa * 16 + b, 16)][0]
                nch = (n + CHKB - 1) // CHKB

                def chunk(c, cur):
                    nc = jnp.minimum(n - c * CHKB, CHKB)
                    pltpu.sync_copy(idx_hbm.at[b, sa, pl.ds(c * CHKB, CHKB)],
                                    ibuf.at[pl.ds(0, CHKB)])
                    pltpu.sync_copy(doff_hbm.at[b, sa, pl.ds(c * CHKB, CHKB)],
                                    dbuf.at[pl.ds(0, CHKB)])

                    def one(e, cur):
                        dval = dbuf[pl.ds(e, 16)][0]
                        rr = dval - r * RROWS
                        ok = (rr >= 0) & (rr < RROWS)

                        @pl.when(ok)
                        def _():
                            ci[pl.ds(cur, 16)] = jnp.full(
                                (16,), ibuf[pl.ds(e, 16)][0], jnp.int32)
                            cd[pl.ds(cur, 16)] = jnp.full(
                                (16,), rr * H, jnp.int32)
                        cur2 = jnp.where(ok, cur + 1, cur)

                        @pl.when(cur2 == DRAIN)
                        def _():
                            drain(DRAIN)
                        return jnp.where(cur2 == DRAIN, 0, cur2)

                    def edge4(g, cur):
                        for u in range(4):
                            cur = one(g * 4 + u, cur)
                        return cur

                    cur = lax.fori_loop(0, nc // 4, edge4, cur)
                    return lax.fori_loop((nc // 4) * 4, nc, one, cur)

                return lax.fori_loop(0, nch, chunk, cur)

            cur = lax.fori_loop(0, 16, sub, jnp.int32(0))

            @pl.when(cur > 0)
            def _():
                drain(cur)

            obase = (hh * N + rbase) * H
            dbase = (hh * N + rbase) * 16

            @pl.when(r < RND - 1)
            def _():
                sz = RROWS * H
                pltpu.sync_copy(asum.at[pl.ds(0, sz)], sum_o.at[pl.ds(obase, sz)])
                pltpu.sync_copy(amin.at[pl.ds(0, sz)], min_o.at[pl.ds(obase, sz)])
                pltpu.sync_copy(amax.at[pl.ds(0, sz)], max_o.at[pl.ds(obase, sz)])
                pltpu.sync_copy(assq.at[pl.ds(0, sz)], ssq_o.at[pl.ds(obase, sz)])
                if with_deg:
                    pltpu.sync_copy(adeg.at[pl.ds(0, RROWS * 16)],
                                    deg_o.at[pl.ds(dbase, RROWS * 16)])

            @pl.when(r == RND - 1)
            def _():
                lastr = BRANGE - (RND - 1) * RROWS
                sz = lastr * H
                pltpu.sync_copy(asum.at[pl.ds(0, sz)], sum_o.at[pl.ds(obase, sz)])
                pltpu.sync_copy(amin.at[pl.ds(0, sz)], min_o.at[pl.ds(obase, sz)])
                pltpu.sync_copy(amax.at[pl.ds(0, sz)], max_o.at[pl.ds(obase, sz)])
                pltpu.sync_copy(assq.at[pl.ds(0, sz)], ssq_o.at[pl.ds(obase, sz)])
                if with_deg:
                    pltpu.sync_copy(adeg.at[pl.ds(0, lastr * 16)],
                                    deg_o.at[pl.ds(dbase, lastr * 16)])
            return 0

        lax.fori_loop(0, RND, rnd, 0)

    return kern(table, bidx, bdoff, bcnt.reshape(NSCW * 16))


def _mm_body(a_ref, w_ref, b_ref, y_ref):
    y_ref[...] = jnp.dot(a_ref[...], w_ref[...],
                         preferred_element_type=jnp.float32) + b_ref[...]


def _matmul_bias(a, w, b, blk):
    M, K = a.shape
    F = w.shape[1]
    return pl.pallas_call(
        _mm_body,
        grid=(M // blk,),
        in_specs=[pl.BlockSpec((blk, K), lambda i: (i, 0)),
                  pl.BlockSpec((K, F), lambda i: (0, 0)),
                  pl.BlockSpec((1, F), lambda i: (0, 0))],
        out_specs=pl.BlockSpec((blk, F), lambda i: (i, 0)),
        out_shape=jax.ShapeDtypeStruct((M, F), jnp.float32),
    )(a, w, b.reshape(1, F))


def _stats4(s0, s1, q0, q1, n0, n1, x0, x1, inv, dmask):
    ssum = s0 + s1
    sssq = q0 + q1
    smin = jnp.minimum(n0, n1)
    smax = jnp.maximum(x0, x1)
    mean = ssum * inv
    std = jnp.sqrt(jax.nn.relu(sssq * inv - mean * mean) + 1e-5)
    return jnp.concatenate([
        mean,
        jnp.where(dmask, smin, 0.0),
        jnp.where(dmask, smax, 0.0),
        std,
    ], axis=1)


def _layer_body(h_ref, deg_ref,
                ss0_ref, ss1_ref, sq0_ref, sq1_ref, sn0_ref, sn1_ref,
                sx0_ref, sx1_ref,
                es0_ref, es1_ref, eq0_ref, eq1_ref, en0_ref, en1_ref,
                ex0_ref, ex1_ref,
                wdst_ref, wsrc_ref, wea_ref, c_ref,
                out_ref, psum_ref, psq_ref):
    deg = deg_ref[...]
    cntc = jnp.maximum(deg, 1.0)
    inv = 1.0 / cntc
    ld = jnp.log(jnp.maximum(deg, 1.0) + 1.0)
    amp = ld / AVG_DEG_LOG
    att = AVG_DEG_LOG / ld
    dmask = deg > 0

    s4 = _stats4(ss0_ref[...], ss1_ref[...], sq0_ref[...], sq1_ref[...],
                 sn0_ref[...], sn1_ref[...], sx0_ref[...], sx1_ref[...],
                 inv, dmask)
    e4 = _stats4(es0_ref[...], es1_ref[...], eq0_ref[...], eq1_ref[...],
                 en0_ref[...], en1_ref[...], ex0_ref[...], ex1_ref[...],
                 inv, dmask)
    hm = jnp.where(dmask, h_ref[...], 0.0)

    p = (jnp.dot(hm, wdst_ref[...], preferred_element_type=jnp.float32)
         + jnp.dot(s4, wsrc_ref[...], preferred_element_type=jnp.float32)
         + jnp.dot(e4, wea_ref[...], preferred_element_type=jnp.float32))
    c = c_ref[...]
    out = ((p[:, :H] + c[0:1, :])
           + amp * (p[:, H:2 * H] + c[1:2, :])
           + att * (p[:, 2 * H:] + c[2:3, :]))
    out_ref[...] = out
    psum_ref[...] = jnp.sum(out, axis=0, keepdims=True)[None]
    psq_ref[...] = jnp.sum(out * out, axis=0, keepdims=True)[None]


def _layer_matmul(h, deg2, ssum, smin, smax, sssq, esum, emin, emax, essq,
                  wdst, wsrc, wea, c):
    nb0 = pl.BlockSpec((NBLK, H), lambda i: (i, 0))
    nb1 = pl.BlockSpec((NBLK, H), lambda i: (i + NNB, 0))
    return pl.pallas_call(
        _layer_body,
        grid=(NNB,),
        in_specs=[nb0,
                  pl.BlockSpec((NBLK, 1), lambda i: (i, 0)),
                  nb0, nb1, nb0, nb1, nb0, nb1, nb0, nb1,
                  nb0, nb1, nb0, nb1, nb0, nb1, nb0, nb1,
                  pl.BlockSpec((H, 3 * H), lambda i: (0, 0)),
                  pl.BlockSpec((4 * H, 3 * H), lambda i: (0, 0)),
                  pl.BlockSpec((4 * H, 3 * H), lambda i: (0, 0)),
                  pl.BlockSpec((3, H), lambda i: (0, 0))],
        out_specs=[nb0,
                   pl.BlockSpec((1, 1, H), lambda i: (i, 0, 0)),
                   pl.BlockSpec((1, 1, H), lambda i: (i, 0, 0))],
        out_shape=[jax.ShapeDtypeStruct((N, H), jnp.float32),
                   jax.ShapeDtypeStruct((NNB, 1, H), jnp.float32),
                   jax.ShapeDtypeStruct((NNB, 1, H), jnp.float32)],
    )(h, deg2, ssum, ssum, sssq, sssq, smin, smin, smax, smax,
      esum, esum, essq, essq, emin, emin, emax, emax, wdst, wsrc, wea, c)


def _bn_body(out_ref, psum_ref, psq_ref, gam_ref, bet_ref, h_ref, hnew_ref):
    mu = jnp.sum(psum_ref[...], axis=0) * (1.0 / N)
    msq = jnp.sum(psq_ref[...], axis=0) * (1.0 / N)
    var = msq - mu * mu
    scale = gam_ref[...] * jax.lax.rsqrt(var + 1e-5)
    hnew_ref[...] = h_ref[...] + jax.nn.relu(
        (out_ref[...] - mu) * scale + bet_ref[...])


def _bn_apply(out, psum, psq, gamma, beta, h):
    nb = pl.BlockSpec((NBLK, H), lambda i: (i, 0))
    return pl.pallas_call(
        _bn_body,
        grid=(NNB,),
        in_specs=[nb,
                  pl.BlockSpec((NNB, 1, H), lambda i: (0, 0, 0)),
                  pl.BlockSpec((NNB, 1, H), lambda i: (0, 0, 0)),
                  pl.BlockSpec((1, H), lambda i: (0, 0)),
                  pl.BlockSpec((1, H), lambda i: (0, 0)),
                  nb],
        out_specs=nb,
        out_shape=jax.ShapeDtypeStruct((N, H), jnp.float32),
    )(out, psum, psq, gamma.reshape(1, H), beta.reshape(1, H), h)


def _head_body(h_ref, batch_ref, w1_ref, b1_ref, w2_ref, b2_ref, w3_ref, b3_ref,
               out_ref):
    gids = jax.lax.broadcasted_iota(jnp.int32, (NG, N), 0)
    onehot = (batch_ref[...] == gids).astype(jnp.float32)
    gsum = jnp.dot(onehot, h_ref[...], preferred_element_type=jnp.float32)
    gcnt = jnp.sum(onehot, axis=1, keepdims=True)
    g = gsum / jnp.maximum(gcnt, 1.0)
    z = jax.nn.relu(jnp.dot(g, w1_ref[...], preferred_element_type=jnp.float32)
                    + b1_ref[...])
    z = jax.nn.relu(jnp.dot(z, w2_ref[...], preferred_element_type=jnp.float32)
                    + b2_ref[...])
    out_ref[...] = jnp.dot(z, w3_ref[...], preferred_element_type=jnp.float32) \
        + b3_ref[...]


def _head(h, batch, fc1_W, fc1_b, fc2_W, fc2_b, fc3_W, fc3_b):
    full = lambda s: pl.BlockSpec(s, lambda: (0,) * len(s))
    return pl.pallas_call(
        _head_body,
        in_specs=[full((N, H)), full((1, N)),
                  full(fc1_W.shape), full((1, fc1_b.shape[0])),
                  full(fc2_W.shape), full((1, fc2_b.shape[0])),
                  full(fc3_W.shape), full((1, fc3_b.shape[0]))],
        out_specs=full((NG, 10)),
        out_shape=jax.ShapeDtypeStruct((NG, 10), jnp.float32),
    )(h, batch.reshape(1, N), fc1_W, fc1_b.reshape(1, -1),
      fc2_W, fc2_b.reshape(1, -1), fc3_W, fc3_b.reshape(1, -1))


def kernel(x, edge_index, batch, edge_attr, W_ne, b_ne, W_ee, b_ee, W_conv,
           b_conv, bn_gamma, bn_beta, fc1_W, fc1_b, fc2_W, fc2_b, fc3_W, fc3_b):
    src, dst = edge_index[0], edge_index[1]
    h = _matmul_bias(x, W_ne, b_ne, NBLK)
    ea = _matmul_bias(edge_attr, W_ee, b_ee, EBLK)

    beid, bsrc, bdoff, bcnt = _bucketize(src, dst)
    esum, emin, emax, essq, degf = _seg_stats(ea, beid, bdoff, bcnt, True)
    degp = degf.reshape(2, N, 16)
    deg2 = degp[0, :, :1] + degp[1, :, :1]
    esum = esum.reshape(2 * N, H)
    emin = emin.reshape(2 * N, H)
    emax = emax.reshape(2 * N, H)
    essq = essq.reshape(2 * N, H)

    wq = W_conv.reshape(NLAYERS, 3, 4, 3, H, H)
    wdst = jnp.transpose(wq[:, :, :3, 0].sum(2), (0, 2, 1, 3)).reshape(NLAYERS, H, 3 * H)
    wsrc = jnp.transpose(wq[:, :, :, 1], (0, 2, 3, 1, 4)).reshape(NLAYERS, 4 * H, 3 * H)
    wea = jnp.transpose(wq[:, :, :, 2], (0, 2, 3, 1, 4)).reshape(NLAYERS, 4 * H, 3 * H)
    cs = jnp.float32(np.sqrt(1e-5)) * wq[:, :, 3, 0].sum(axis=2)
    cs = cs.at[:, 0].add(b_conv)

    for i in range(NLAYERS):
        ssum, smin, smax, sssq = _seg_stats(h, bsrc, bdoff, bcnt, False)
        out, psum, psq = _layer_matmul(h, deg2,
                                       ssum.reshape(2 * N, H),
                                       smin.reshape(2 * N, H),
                                       smax.reshape(2 * N, H),
                                       sssq.reshape(2 * N, H),
                                       esum, emin, emax, essq,
                                       wdst[i], wsrc[i], wea[i], cs[i])
        h = _bn_apply(out, psum, psq, bn_gamma[i], bn_beta[i], h)

    return _head(h, batch, fc1_W, fc1_b, fc2_W, fc2_b, fc3_W, fc3_b)
